# baseline (speedup 1.0000x reference)
import functools

import jax
import jax.numpy as jnp
import numpy as np
from jax.experimental import pallas as pl


def _mm_kernel(x_ref, w_ref, b_ref, o_ref, *, act):
    acc = jnp.dot(x_ref[...], w_ref[...], preferred_element_type=jnp.float32)
    acc = acc + b_ref[...]
    if act == "relu":
        acc = jnp.maximum(acc, 0.0)
    o_ref[...] = acc


def _mm(x, w, b, act=None, bm=400):
    m, k = x.shape
    n = w.shape[1]
    if m % bm != 0:
        bm = m
    b2 = b.reshape(1, n)
    return pl.pallas_call(
        functools.partial(_mm_kernel, act=act),
        out_shape=jax.ShapeDtypeStruct((m, n), jnp.float32),
        grid=(m // bm,),
        in_specs=[
            pl.BlockSpec((bm, k), lambda i: (i, 0)),
            pl.BlockSpec((k, n), lambda i: (0, 0)),
            pl.BlockSpec((1, n), lambda i: (0, 0)),
        ],
        out_specs=pl.BlockSpec((bm, n), lambda i: (i, 0)),
    )(x, w, b2)


def _attn_kernel(q_ref, k_ref, v_ref, bq_ref, bk_ref, o_ref, *, scale):
    q = q_ref[0]
    k = k_ref[0]
    v = v_ref[0]
    s = jax.lax.dot_general(
        q, k, (((1,), (1,)), ((), ())), preferred_element_type=jnp.float32
    ) * scale
    same = bq_ref[...] == bk_ref[...]
    s = jnp.where(same, s, -1e9)
    m = jnp.max(s, axis=-1, keepdims=True)
    p = jnp.exp(s - m)
    out = jnp.dot(p, v, preferred_element_type=jnp.float32)
    out = out / jnp.sum(p, axis=-1, keepdims=True)
    o_ref[0] = out


def _attention(qh, kh, vh, batch, bq=400):
    H, n, dh = qh.shape
    bcol = batch.reshape(n, 1)
    brow = batch.reshape(1, n)
    return pl.pallas_call(
        functools.partial(_attn_kernel, scale=1.0 / np.sqrt(dh)),
        out_shape=jax.ShapeDtypeStruct((H, n, dh), jnp.float32),
        grid=(H, n // bq),
        in_specs=[
            pl.BlockSpec((1, bq, dh), lambda h, i: (h, i, 0)),
            pl.BlockSpec((1, n, dh), lambda h, i: (h, 0, 0)),
            pl.BlockSpec((1, n, dh), lambda h, i: (h, 0, 0)),
            pl.BlockSpec((bq, 1), lambda h, i: (i, 0)),
            pl.BlockSpec((1, n), lambda h, i: (0, 0)),
        ],
        out_specs=pl.BlockSpec((1, bq, dh), lambda h, i: (h, i, 0)),
    )(qh, kh, vh, bcol, brow)


def _ln(t, g, b):
    mu = jnp.mean(t, axis=-1, keepdims=True)
    var = jnp.mean((t - mu) ** 2, axis=-1, keepdims=True)
    return (t - mu) * jax.lax.rsqrt(var + 1e-5) * g + b


def _fusion_kernel(c_ref, o_ref, ln1g_ref, ln1b_ref, w1_ref, b1_ref,
                   w2_ref, b2_ref, ln2g_ref, ln2b_ref, f_ref):
    h1 = _ln(c_ref[...] + o_ref[...], ln1g_ref[...], ln1b_ref[...])
    t = jnp.maximum(
        jnp.dot(h1, w1_ref[...], preferred_element_type=jnp.float32)
        + b1_ref[...], 0.0)
    t = jnp.dot(t, w2_ref[...], preferred_element_type=jnp.float32) + b2_ref[...]
    f_ref[...] = _ln(h1 + t, ln2g_ref[...], ln2b_ref[...])


def _fusion(c, o, ln1_g, ln1_b, W1, b1, W2, b2, ln2_g, ln2_b, bm=400):
    n, p = c.shape
    ffn = W1.shape[1]
    row = lambda a, d: a.reshape(1, d)
    return pl.pallas_call(
        _fusion_kernel,
        out_shape=jax.ShapeDtypeStruct((n, p), jnp.float32),
        grid=(n // bm,),
        in_specs=[
            pl.BlockSpec((bm, p), lambda i: (i, 0)),
            pl.BlockSpec((bm, p), lambda i: (i, 0)),
            pl.BlockSpec((1, p), lambda i: (0, 0)),
            pl.BlockSpec((1, p), lambda i: (0, 0)),
            pl.BlockSpec((p, ffn), lambda i: (0, 0)),
            pl.BlockSpec((1, ffn), lambda i: (0, 0)),
            pl.BlockSpec((ffn, p), lambda i: (0, 0)),
            pl.BlockSpec((1, p), lambda i: (0, 0)),
            pl.BlockSpec((1, p), lambda i: (0, 0)),
            pl.BlockSpec((1, p), lambda i: (0, 0)),
        ],
        out_specs=pl.BlockSpec((bm, p), lambda i: (i, 0)),
    )(c, o, row(ln1_g, p), row(ln1_b, p), W1, row(b1, ffn), W2,
      row(b2, p), row(ln2_g, p), row(ln2_b, p))


def kernel(x, edge_index, edge_type, batch, Wp, bp, W_rel, W_root, b_rg,
           Wc, bc, Wg, bg, Wq, Wk, Wv, Wo, bo, ln1_g, ln1_b, W1, b1,
           W2, b2, ln2_g, ln2_b, Wf, bf, Wcls, bcls):
    n = x.shape[0]
    L, R, HID, _ = W_rel.shape
    src = edge_index[0]
    dst = edge_index[1]

    seg = dst * R + edge_type
    cnt = jax.ops.segment_sum(jnp.ones_like(seg, jnp.float32), seg,
                              num_segments=n * R)
    inv_cnt = 1.0 / jnp.clip(cnt, 1.0)

    h = _mm(x, Wp, bp)
    for l in range(L):
        agg = jax.ops.segment_sum(h.astype(jnp.bfloat16)[src], seg,
                                  num_segments=n * R)
        mean = (agg.astype(jnp.float32) * inv_cnt[:, None]).reshape(n, R * HID)
        feats = jnp.concatenate([h, mean], axis=1)
        Wcat = jnp.concatenate(
            [W_root[l], W_rel[l, 0], W_rel[l, 1], W_rel[l, 2]], axis=0)
        h = _mm(feats, Wcat, b_rg[l], act="relu")

    P = Wc.shape[1]
    H = 4
    dh = P // H
    c = _mm(x, Wc, bc)
    g = _mm(h, Wg, bg)
    q = _mm(c, Wq, jnp.zeros((P,), jnp.float32))
    k = _mm(g, Wk, jnp.zeros((P,), jnp.float32))
    v = _mm(g, Wv, jnp.zeros((P,), jnp.float32))
    qh = q.reshape(n, H, dh).transpose(1, 0, 2)
    kh = k.reshape(n, H, dh).transpose(1, 0, 2)
    vh = v.reshape(n, H, dh).transpose(1, 0, 2)
    oh = _attention(qh, kh, vh, batch)
    o = _mm(oh.transpose(1, 0, 2).reshape(n, P), Wo, bo)
    f = _fusion(c, o, ln1_g, ln1_b, W1, b1, W2, b2, ln2_g, ln2_b)

    counts = jax.ops.segment_sum(jnp.ones((n,), jnp.float32), batch,
                                 num_segments=64)
    pooled = jax.ops.segment_sum(f, batch, num_segments=64)
    pooled = pooled / jnp.clip(counts, 1.0)[:, None]
    graph_repr = _mm(pooled, Wf, bf, bm=64)
    logits = _mm(graph_repr, Wcls, bcls, bm=64)
    return logits


# speedup vs baseline: 1.3837x; 1.3837x over previous
import functools

import jax
import jax.numpy as jnp
import numpy as np
from jax.experimental import pallas as pl


def _mm_kernel(x_ref, w_ref, b_ref, o_ref, *, act):
    acc = jnp.dot(x_ref[...], w_ref[...], preferred_element_type=jnp.float32)
    acc = acc + b_ref[...]
    if act == "relu":
        acc = jnp.maximum(acc, 0.0)
    o_ref[...] = acc


def _mm(x, w, b, act=None, bm=400):
    m, k = x.shape
    n = w.shape[1]
    if m % bm != 0:
        bm = m
    b2 = b.reshape(1, n)
    return pl.pallas_call(
        functools.partial(_mm_kernel, act=act),
        out_shape=jax.ShapeDtypeStruct((m, n), jnp.float32),
        grid=(m // bm,),
        in_specs=[
            pl.BlockSpec((bm, k), lambda i: (i, 0)),
            pl.BlockSpec((k, n), lambda i: (0, 0)),
            pl.BlockSpec((1, n), lambda i: (0, 0)),
        ],
        out_specs=pl.BlockSpec((bm, n), lambda i: (i, 0)),
    )(x, w, b2)


def _attn_kernel(q_ref, k_ref, v_ref, bq_ref, bk_ref, o_ref, *, scale):
    q = q_ref[0]
    k = k_ref[0]
    v = v_ref[0]
    s = jax.lax.dot_general(
        q, k, (((1,), (1,)), ((), ())), preferred_element_type=jnp.float32
    ) * scale
    same = bq_ref[...] == bk_ref[...]
    s = jnp.where(same, s, -1e9)
    m = jnp.max(s, axis=-1, keepdims=True)
    p = jnp.exp(s - m)
    out = jnp.dot(p, v, preferred_element_type=jnp.float32)
    out = out / jnp.sum(p, axis=-1, keepdims=True)
    o_ref[0] = out


def _attention(qh, kh, vh, batch, bq=400):
    H, n, dh = qh.shape
    bcol = batch.reshape(n, 1)
    brow = batch.reshape(1, n)
    return pl.pallas_call(
        functools.partial(_attn_kernel, scale=1.0 / np.sqrt(dh)),
        out_shape=jax.ShapeDtypeStruct((H, n, dh), jnp.float32),
        grid=(H, n // bq),
        in_specs=[
            pl.BlockSpec((1, bq, dh), lambda h, i: (h, i, 0)),
            pl.BlockSpec((1, n, dh), lambda h, i: (h, 0, 0)),
            pl.BlockSpec((1, n, dh), lambda h, i: (h, 0, 0)),
            pl.BlockSpec((bq, 1), lambda h, i: (i, 0)),
            pl.BlockSpec((1, n), lambda h, i: (0, 0)),
        ],
        out_specs=pl.BlockSpec((1, bq, dh), lambda h, i: (h, i, 0)),
    )(qh, kh, vh, bcol, brow)


def _ln(t, g, b):
    mu = jnp.mean(t, axis=-1, keepdims=True)
    var = jnp.mean((t - mu) ** 2, axis=-1, keepdims=True)
    return (t - mu) * jax.lax.rsqrt(var + 1e-5) * g + b


def _fusion_kernel(c_ref, o_ref, ln1g_ref, ln1b_ref, w1_ref, b1_ref,
                   w2_ref, b2_ref, ln2g_ref, ln2b_ref, f_ref):
    h1 = _ln(c_ref[...] + o_ref[...], ln1g_ref[...], ln1b_ref[...])
    t = jnp.maximum(
        jnp.dot(h1, w1_ref[...], preferred_element_type=jnp.float32)
        + b1_ref[...], 0.0)
    t = jnp.dot(t, w2_ref[...], preferred_element_type=jnp.float32) + b2_ref[...]
    f_ref[...] = _ln(h1 + t, ln2g_ref[...], ln2b_ref[...])


def _fusion(c, o, ln1_g, ln1_b, W1, b1, W2, b2, ln2_g, ln2_b, bm=400):
    n, p = c.shape
    ffn = W1.shape[1]
    row = lambda a, d: a.reshape(1, d)
    return pl.pallas_call(
        _fusion_kernel,
        out_shape=jax.ShapeDtypeStruct((n, p), jnp.float32),
        grid=(n // bm,),
        in_specs=[
            pl.BlockSpec((bm, p), lambda i: (i, 0)),
            pl.BlockSpec((bm, p), lambda i: (i, 0)),
            pl.BlockSpec((1, p), lambda i: (0, 0)),
            pl.BlockSpec((1, p), lambda i: (0, 0)),
            pl.BlockSpec((p, ffn), lambda i: (0, 0)),
            pl.BlockSpec((1, ffn), lambda i: (0, 0)),
            pl.BlockSpec((ffn, p), lambda i: (0, 0)),
            pl.BlockSpec((1, p), lambda i: (0, 0)),
            pl.BlockSpec((1, p), lambda i: (0, 0)),
            pl.BlockSpec((1, p), lambda i: (0, 0)),
        ],
        out_specs=pl.BlockSpec((bm, p), lambda i: (i, 0)),
    )(c, o, row(ln1_g, p), row(ln1_b, p), W1, row(b1, ffn), W2,
      row(b2, p), row(ln2_g, p), row(ln2_b, p))


def kernel(x, edge_index, edge_type, batch, Wp, bp, W_rel, W_root, b_rg,
           Wc, bc, Wg, bg, Wq, Wk, Wv, Wo, bo, ln1_g, ln1_b, W1, b1,
           W2, b2, ln2_g, ln2_b, Wf, bf, Wcls, bcls):
    n = x.shape[0]
    L, R, HID, _ = W_rel.shape
    src = edge_index[0]
    dst = edge_index[1]

    seg = dst * R + edge_type
    cnt = jax.ops.segment_sum(jnp.ones_like(seg, jnp.float32), seg,
                              num_segments=n * R)
    inv_cnt = 1.0 / jnp.clip(cnt, 1.0)

    h = _mm(x, Wp, bp)
    for l in range(L):
        agg = jax.ops.segment_sum(h[src], seg, num_segments=n * R)
        mean = (agg * inv_cnt[:, None]).reshape(n, R * HID)
        feats = jnp.concatenate([h, mean], axis=1)
        Wcat = jnp.concatenate(
            [W_root[l], W_rel[l, 0], W_rel[l, 1], W_rel[l, 2]], axis=0)
        h = _mm(feats, Wcat, b_rg[l], act="relu")

    P = Wc.shape[1]
    H = 4
    dh = P // H
    c = _mm(x, Wc, bc)
    g = _mm(h, Wg, bg)
    q = _mm(c, Wq, jnp.zeros((P,), jnp.float32))
    k = _mm(g, Wk, jnp.zeros((P,), jnp.float32))
    v = _mm(g, Wv, jnp.zeros((P,), jnp.float32))
    qh = q.reshape(n, H, dh).transpose(1, 0, 2)
    kh = k.reshape(n, H, dh).transpose(1, 0, 2)
    vh = v.reshape(n, H, dh).transpose(1, 0, 2)
    oh = _attention(qh, kh, vh, batch)
    o = _mm(oh.transpose(1, 0, 2).reshape(n, P), Wo, bo)
    f = _fusion(c, o, ln1_g, ln1_b, W1, b1, W2, b2, ln2_g, ln2_b)

    counts = jax.ops.segment_sum(jnp.ones((n,), jnp.float32), batch,
                                 num_segments=64)
    pooled = jax.ops.segment_sum(f, batch, num_segments=64)
    pooled = pooled / jnp.clip(counts, 1.0)[:, None]
    graph_repr = _mm(pooled, Wf, bf, bm=64)
    logits = _mm(graph_repr, Wcls, bcls, bm=64)
    return logits
